# baseline (speedup 1.0000x reference)
import functools

import jax
import jax.numpy as jnp
from jax import lax
from jax.experimental import pallas as pl
from jax.experimental.pallas import tpu as pltpu
from jax.experimental.pallas import tpu_sc as plsc

_N = 1000000
_D = 16
_B = 4096
_REG = 0.05

_NC = 2
_NS = 16
_NW = _NC * _NS
_CHUNKS = (2 * _B) // (_NW * 128)


def _sc_box_loss(nf1_rows, dis_rows, table):
    mesh = plsc.VectorSubcoreMesh(core_axis_name="c", subcore_axis_name="s")

    @functools.partial(
        pl.kernel,
        out_type=jax.ShapeDtypeStruct((_NW, 2, 16), jnp.float32),
        mesh=mesh,
        compiler_params=pltpu.CompilerParams(use_tc_tiling_on_sc=False),
        scratch_types=[
            pltpu.VMEM((_CHUNKS, 128), jnp.int32),
            pltpu.VMEM((_CHUNKS, 128), jnp.int32),
            pltpu.VMEM((_CHUNKS, 128, 2 * _D), jnp.float32),
            pltpu.VMEM((_CHUNKS, 128, 2 * _D), jnp.float32),
            pltpu.VMEM((2, 16), jnp.float32),
            pltpu.SemaphoreType.DMA,
        ],
    )
    def k(nf1_hbm, dis_hbm, table_hbm, out_hbm, idx1, idx2, rows1, rows2, outv, sem):
        wid = lax.axis_index("s") * _NC + lax.axis_index("c")
        base = wid * _CHUNKS
        pltpu.sync_copy(nf1_hbm.at[pl.ds(base, _CHUNKS)], idx1)
        pltpu.sync_copy(dis_hbm.at[pl.ds(base, _CHUNKS)], idx2)
        copies = []
        for j in range(_CHUNKS):
            copies.append(pltpu.async_copy(table_hbm.at[idx1.at[j]], rows1.at[j], sem))
        for j in range(_CHUNKS):
            copies.append(pltpu.async_copy(table_hbm.at[idx2.at[j]], rows2.at[j], sem))
        for c in copies:
            c.wait()

        def body(p, accs):
            acc_in, acc_dis = accs
            def boxes(rows, j, r):
                c = rows[j, r, pl.ds(0, _D)]
                o = jnp.abs(rows[j, r, pl.ds(_D, _D)])
                return c, o

            for j in range(_CHUNKS):
                c1, o1 = boxes(rows1, j, 2 * p)
                c2, o2 = boxes(rows1, j, 2 * p + 1)
                t = jnp.maximum(jnp.abs(c1 - c2) + o1 - o2, 0.0)
                acc_in = acc_in + t * t
                d1, p1 = boxes(rows2, j, 2 * p)
                d2, p2 = boxes(rows2, j, 2 * p + 1)
                u = jnp.maximum(p1 + p2 - jnp.abs(d1 - d2), 0.0)
                acc_dis = acc_dis + u * u
            return acc_in, acc_dis

        zeros = jnp.zeros((16,), jnp.float32)
        acc_in, acc_dis = lax.fori_loop(0, 64, body, (zeros, zeros))
        outv[0, :] = acc_in
        outv[1, :] = acc_dis
        pltpu.sync_copy(outv, out_hbm.at[wid])

    return k(nf1_rows, dis_rows, table)


def _tc_norm_sum(bumps128):
    rows, lanes = bumps128.shape
    blk = 5000
    grid = rows // blk

    def body(x_ref, o_ref):
        x = x_ref[...]
        z = x * x
        r = lax.broadcasted_iota(jnp.int32, (128, 8), 0)
        c = lax.broadcasted_iota(jnp.int32, (128, 8), 1)
        sel = jnp.where(r // 16 == c, 1.0, 0.0).astype(jnp.float32)
        seg = lax.dot_general(z, sel, (((1,), (0,)), ((), ())),
                              preferred_element_type=jnp.float32)
        part = jnp.sum(jnp.sqrt(seg))

        @pl.when(pl.program_id(0) == 0)
        def _():
            o_ref[...] = jnp.zeros_like(o_ref)

        o_ref[...] += part

    return pl.pallas_call(
        body,
        grid=(grid,),
        in_specs=[pl.BlockSpec((blk, lanes), lambda i: (i, 0))],
        out_specs=pl.BlockSpec((1, 1), lambda i: (0, 0)),
        out_shape=jax.ShapeDtypeStruct((1, 1), jnp.float32),
    )(bumps128)


def kernel(nf1, disjoint, class_embeds, bumps):
    nf1_rows = nf1.reshape(_NW * _CHUNKS, 128)
    dis_rows = disjoint.reshape(_NW * _CHUNKS, 128)
    parts = _sc_box_loss(nf1_rows, dis_rows, class_embeds)
    nsum = _tc_norm_sum(bumps.reshape(_N // 8, 128))
    incl = jnp.sum(parts[:, 0, :]) / _B
    dis = jnp.sum(parts[:, 1, :]) / _B
    return incl + dis + _REG * nsum[0, 0] / _N


# speedup vs baseline: 1.6044x; 1.6044x over previous
import functools

import jax
import jax.numpy as jnp
from jax import lax
from jax.experimental import pallas as pl
from jax.experimental.pallas import tpu as pltpu
from jax.experimental.pallas import tpu_sc as plsc

_N = 1000000
_D = 16
_B = 4096
_REG = 0.05

_NC = 2
_NS = 16
_NW = _NC * _NS
_CHUNKS = (2 * _B) // (_NW * 128)


def _sc_box_loss(nf1_rows, dis_rows, table):
    mesh = plsc.VectorSubcoreMesh(core_axis_name="c", subcore_axis_name="s")

    @functools.partial(
        pl.kernel,
        out_type=jax.ShapeDtypeStruct((_NW, 2, 16), jnp.float32),
        mesh=mesh,
        compiler_params=pltpu.CompilerParams(use_tc_tiling_on_sc=False),
        scratch_types=[
            pltpu.VMEM((_CHUNKS, 128), jnp.int32),
            pltpu.VMEM((_CHUNKS, 128), jnp.int32),
            pltpu.VMEM((_CHUNKS, 128, 2 * _D), jnp.float32),
            pltpu.VMEM((_CHUNKS, 128, 2 * _D), jnp.float32),
            pltpu.VMEM((2, 16), jnp.float32),
            pltpu.SemaphoreType.DMA,
        ],
    )
    def k(nf1_hbm, dis_hbm, table_hbm, out_hbm, idx1, idx2, rows1, rows2, outv, sem):
        wid = lax.axis_index("s") * _NC + lax.axis_index("c")
        base = wid * _CHUNKS
        pltpu.sync_copy(nf1_hbm.at[pl.ds(base, _CHUNKS)], idx1)
        pltpu.sync_copy(dis_hbm.at[pl.ds(base, _CHUNKS)], idx2)
        copies = []
        for j in range(_CHUNKS):
            copies.append(pltpu.async_copy(table_hbm.at[idx1.at[j]], rows1.at[j], sem))
        for j in range(_CHUNKS):
            copies.append(pltpu.async_copy(table_hbm.at[idx2.at[j]], rows2.at[j], sem))
        for c in copies:
            c.wait()

        def body(p, accs):
            acc_in, acc_dis = accs

            def boxes(rows, j, r):
                c = rows[j, r, pl.ds(0, _D)]
                o = jnp.abs(rows[j, r, pl.ds(_D, _D)])
                return c, o

            for j in range(_CHUNKS):
                c1, o1 = boxes(rows1, j, 2 * p)
                c2, o2 = boxes(rows1, j, 2 * p + 1)
                t = jnp.maximum(jnp.abs(c1 - c2) + o1 - o2, 0.0)
                acc_in = acc_in + t * t
                d1, p1 = boxes(rows2, j, 2 * p)
                d2, p2 = boxes(rows2, j, 2 * p + 1)
                u = jnp.maximum(p1 + p2 - jnp.abs(d1 - d2), 0.0)
                acc_dis = acc_dis + u * u
            return acc_in, acc_dis

        zeros = jnp.zeros((16,), jnp.float32)
        acc_in, acc_dis = lax.fori_loop(0, 64, body, (zeros, zeros))
        outv[0, :] = acc_in
        outv[1, :] = acc_dis
        pltpu.sync_copy(outv, out_hbm.at[wid])

    return k(nf1_rows, dis_rows, table)


def _tc_norm_sum(bumps_t):
    d, n = bumps_t.shape
    blk = 25600
    grid = (n + blk - 1) // blk

    def body(x_ref, o_ref):
        i = pl.program_id(0)
        x = x_ref[...]
        gidx = i * blk + lax.broadcasted_iota(jnp.int32, (d, blk), 1)
        xm = jnp.where(gidx < n, x, 0.0)
        s = jnp.sum(xm * xm, axis=0)
        part = jnp.sum(jnp.sqrt(s))

        @pl.when(i == 0)
        def _():
            o_ref[...] = jnp.zeros_like(o_ref)

        o_ref[...] += part

    return pl.pallas_call(
        body,
        grid=(grid,),
        in_specs=[pl.BlockSpec((d, blk), lambda i: (0, i))],
        out_specs=pl.BlockSpec((1, 1), lambda i: (0, 0)),
        out_shape=jax.ShapeDtypeStruct((1, 1), jnp.float32),
    )(bumps_t)


def kernel(nf1, disjoint, class_embeds, bumps):
    nf1_rows = nf1.reshape(_NW * _CHUNKS, 128)
    dis_rows = disjoint.reshape(_NW * _CHUNKS, 128)
    parts = _sc_box_loss(nf1_rows, dis_rows, class_embeds)
    nsum = _tc_norm_sum(bumps.T)
    incl = jnp.sum(parts[:, 0, :]) / _B
    dis = jnp.sum(parts[:, 1, :]) / _B
    return incl + dis + _REG * nsum[0, 0] / _N


# speedup vs baseline: 1.6347x; 1.0189x over previous
import functools

import jax
import jax.numpy as jnp
from jax import lax
from jax.experimental import pallas as pl
from jax.experimental.pallas import tpu as pltpu
from jax.experimental.pallas import tpu_sc as plsc

_N = 1000000
_D = 16
_B = 4096
_REG = 0.05

_NC = 2
_NS = 16
_NW = _NC * _NS
_CPW = 4
_PACK = 4


def _sc_box_loss(nf1_rows, dis_rows, table4):
    mesh = plsc.VectorSubcoreMesh(core_axis_name="c", subcore_axis_name="s")

    @functools.partial(
        pl.kernel,
        out_type=jax.ShapeDtypeStruct((_NW, 2, 16), jnp.float32),
        mesh=mesh,
        compiler_params=pltpu.CompilerParams(use_tc_tiling_on_sc=False),
        scratch_types=[
            pltpu.VMEM((_CPW, 128), jnp.int32),
            pltpu.VMEM((_CPW, 128), jnp.int32),
            pltpu.VMEM((_CPW, 128, 128), jnp.float32),
            pltpu.VMEM((2, 16), jnp.float32),
            pltpu.SemaphoreType.DMA,
        ],
    )
    def k(nf1_hbm, dis_hbm, tab_hbm, out_hbm, idxv, ridx, rows, outv, sem):
        wid = lax.axis_index("s") * _NC + lax.axis_index("c")
        base = wid * 2
        for k2 in range(2):
            pltpu.sync_copy(nf1_hbm.at[base + k2], idxv.at[k2])
            pltpu.sync_copy(dis_hbm.at[base + k2], idxv.at[2 + k2])
        for k2 in range(_CPW):
            for m in range(8):
                sl = pl.ds(16 * m, 16)
                ridx[k2, sl] = idxv[k2, sl] >> 2
        copies = []
        for k2 in range(_CPW):
            copies.append(pltpu.async_copy(tab_hbm.at[ridx.at[k2]], rows.at[k2], sem))
        for c in copies:
            c.wait()

        def make_body(k2):
            def body(p, acc):
                ia = idxv[k2, pl.ds(2 * p, 1)][0]
                ib = idxv[k2, pl.ds(2 * p + 1, 1)][0]
                offa = (ia & (_PACK - 1)) * (2 * _D)
                offb = (ib & (_PACK - 1)) * (2 * _D)
                c1 = rows[k2, 2 * p, pl.ds(offa, _D)]
                o1 = jnp.abs(rows[k2, 2 * p, pl.ds(offa + _D, _D)])
                c2 = rows[k2, 2 * p + 1, pl.ds(offb, _D)]
                o2 = jnp.abs(rows[k2, 2 * p + 1, pl.ds(offb + _D, _D)])
                d = jnp.abs(c1 - c2)
                if k2 < 2:
                    t = jnp.maximum(d + o1 - o2, 0.0)
                else:
                    t = jnp.maximum(o1 + o2 - d, 0.0)
                return acc + t * t

            return body

        zeros = jnp.zeros((16,), jnp.float32)
        acc_in = lax.fori_loop(0, 64, make_body(0), zeros)
        acc_in = lax.fori_loop(0, 64, make_body(1), acc_in)
        acc_dis = lax.fori_loop(0, 64, make_body(2), zeros)
        acc_dis = lax.fori_loop(0, 64, make_body(3), acc_dis)
        outv[0, :] = acc_in
        outv[1, :] = acc_dis
        pltpu.sync_copy(outv, out_hbm.at[wid])

    return k(nf1_rows, dis_rows, table4)


def _tc_norm_sum(bumps_t):
    d, n = bumps_t.shape
    blk = 65536
    grid = (n + blk - 1) // blk

    def body(x_ref, o_ref):
        i = pl.program_id(0)
        x = x_ref[...]
        s = jnp.sum(x * x, axis=0, keepdims=True)
        gidx = i * blk + lax.broadcasted_iota(jnp.int32, (1, blk), 1)
        s = jnp.where(gidx < n, s, 0.0)
        part = jnp.sum(jnp.sqrt(s))

        @pl.when(i == 0)
        def _():
            o_ref[...] = jnp.zeros_like(o_ref)

        o_ref[...] += part

    return pl.pallas_call(
        body,
        grid=(grid,),
        in_specs=[pl.BlockSpec((d, blk), lambda i: (0, i))],
        out_specs=pl.BlockSpec((1, 1), lambda i: (0, 0)),
        out_shape=jax.ShapeDtypeStruct((1, 1), jnp.float32),
    )(bumps_t)


def kernel(nf1, disjoint, class_embeds, bumps):
    nf1_rows = nf1.reshape(_NW * 2, 128)
    dis_rows = disjoint.reshape(_NW * 2, 128)
    table4 = class_embeds.reshape(_N * 2 * _D // 128, 128)
    parts = _sc_box_loss(nf1_rows, dis_rows, table4)
    nsum = _tc_norm_sum(bumps.T)
    incl = jnp.sum(parts[:, 0, :]) / _B
    dis = jnp.sum(parts[:, 1, :]) / _B
    return incl + dis + _REG * nsum[0, 0] / _N
